# baseline (speedup 1.0000x reference)
import functools

import jax
import jax.numpy as jnp
from jax import lax
from jax.experimental import pallas as pl
from jax.experimental.pallas import tpu as pltpu
from jax.experimental.pallas import tpu_sc as plsc

_VOCAB = 1_000_000
_DIM = 64
_B = 4096
_L = 200

_NC = 2
_NS = 16
_NW = _NC * _NS
_TOT = _B * _L
_BPW = _TOT // _NW
_CH = 128
_NCH = _BPW // _CH
_NB = 8
_DW = 2
_DG = _NB - _DW


def _emb_body(idx_hbm, table_hbm, out_hbm, idx_v, rows_v, gsem, wsem):
    wid = lax.axis_index("s") * _NC + lax.axis_index("c")
    base = wid * _BPW

    pltpu.sync_copy(idx_hbm.at[wid], idx_v)

    def start_gather(g, slot):
        pltpu.async_copy(table_hbm.at[idx_v.at[g]], rows_v.at[slot], gsem)

    def wait_gather(g, slot):
        pltpu.make_async_copy(
            table_hbm.at[idx_v.at[g]], rows_v.at[slot], gsem).wait()

    def start_wb(g, slot):
        pltpu.async_copy(
            rows_v.at[slot], out_hbm.at[pl.ds(base + g * _CH, _CH)], wsem)

    def wait_wb(g, slot):
        pltpu.make_async_copy(
            rows_v.at[slot], out_hbm.at[pl.ds(base + g * _CH, _CH)],
            wsem).wait()

    for g in range(_DG):
        start_gather(g, g)

    def step(g, b):
        wait_gather(g, b)
        start_wb(g, b)
        wait_wb(g - _DW, (b - _DW) % _NB)
        start_gather(g + _DG, (b + _DG) % _NB)

    for g in range(_NB):
        wait_gather(g, g)
        start_wb(g, g)
        if g >= _DW:
            wait_wb(g - _DW, (g - _DW) % _NB)
        start_gather(g + _DG, (g + _DG) % _NB)

    @pl.loop(_NB, _NCH - _NB, step=_NB)
    def _steady(go):
        for b in range(_NB):
            step(go + b, b)

    for g in range(_NCH - _NB, _NCH):
        wait_gather(g, g % _NB)
        start_wb(g, g % _NB)
        wait_wb(g - _DW, (g - _DW) % _NB)
        if g + _DG < _NCH:
            start_gather(g + _DG, (g + _DG) % _NB)

    for g in range(_NCH - _DW, _NCH):
        wait_wb(g, g % _NB)


@jax.jit
def _embedding_lookup(idx, table):
    mesh = plsc.VectorSubcoreMesh(core_axis_name="c", subcore_axis_name="s")
    fn = functools.partial(
        pl.kernel,
        out_type=jax.ShapeDtypeStruct((_TOT, _DIM), jnp.float32),
        mesh=mesh,
        scratch_types=[
            pltpu.VMEM((_NCH, _CH), jnp.int32),
            pltpu.VMEM((_NB, _CH, _DIM), jnp.float32),
            pltpu.SemaphoreType.DMA,
            pltpu.SemaphoreType.DMA,
        ],
        compiler_params=pltpu.CompilerParams(use_tc_tiling_on_sc=False),
    )(_emb_body)
    return fn(idx, table)


def kernel(inputs, table):
    idx = inputs.reshape(_NW, _NCH, _CH).astype(jnp.int32)
    out = _embedding_lookup(idx, table)
    return out.reshape(_B, _L, _DIM)


# speedup vs baseline: 1.0010x; 1.0010x over previous
import functools

import jax
import jax.numpy as jnp
from jax import lax
from jax.experimental import pallas as pl
from jax.experimental.pallas import tpu as pltpu
from jax.experimental.pallas import tpu_sc as plsc

_VOCAB = 1_000_000
_DIM = 64
_B = 4096
_L = 200

_NC = 2
_NS = 16
_NW = _NC * _NS
_TOT = _B * _L
_BPW = _TOT // _NW
_CH = 256
_NCH = _BPW // _CH
_NB = 4
_DW = 1
_DG = _NB - _DW


def _emb_body(idx_hbm, table_hbm, out_hbm, idx_v, rows_v, gsem, wsem):
    wid = lax.axis_index("s") * _NC + lax.axis_index("c")
    base = wid * _BPW

    pltpu.sync_copy(idx_hbm.at[wid], idx_v)

    def start_gather(g, slot):
        pltpu.async_copy(table_hbm.at[idx_v.at[g]], rows_v.at[slot], gsem)

    def wait_gather(g, slot):
        pltpu.make_async_copy(
            table_hbm.at[idx_v.at[g]], rows_v.at[slot], gsem).wait()

    def start_wb(g, slot):
        pltpu.async_copy(
            rows_v.at[slot], out_hbm.at[pl.ds(base + g * _CH, _CH)], wsem)

    def wait_wb(g, slot):
        pltpu.make_async_copy(
            rows_v.at[slot], out_hbm.at[pl.ds(base + g * _CH, _CH)],
            wsem).wait()

    for g in range(_DG):
        start_gather(g, g)

    def step(g, b):
        wait_gather(g, b)
        start_wb(g, b)
        wait_wb(g - _DW, (b - _DW) % _NB)
        start_gather(g + _DG, (b + _DG) % _NB)

    for g in range(_NB):
        wait_gather(g, g)
        start_wb(g, g)
        if g >= _DW:
            wait_wb(g - _DW, (g - _DW) % _NB)
        start_gather(g + _DG, (g + _DG) % _NB)

    @pl.loop(_NB, _NCH - _NB, step=_NB)
    def _steady(go):
        for b in range(_NB):
            step(go + b, b)

    for g in range(_NCH - _NB, _NCH):
        wait_gather(g, g % _NB)
        start_wb(g, g % _NB)
        wait_wb(g - _DW, (g - _DW) % _NB)
        if g + _DG < _NCH:
            start_gather(g + _DG, (g + _DG) % _NB)

    for g in range(_NCH - _DW, _NCH):
        wait_wb(g, g % _NB)


@jax.jit
def _embedding_lookup(idx, table):
    mesh = plsc.VectorSubcoreMesh(core_axis_name="c", subcore_axis_name="s")
    fn = functools.partial(
        pl.kernel,
        out_type=jax.ShapeDtypeStruct((_TOT, _DIM), jnp.float32),
        mesh=mesh,
        scratch_types=[
            pltpu.VMEM((_NCH, _CH), jnp.int32),
            pltpu.VMEM((_NB, _CH, _DIM), jnp.float32),
            pltpu.SemaphoreType.DMA,
            pltpu.SemaphoreType.DMA,
        ],
        compiler_params=pltpu.CompilerParams(use_tc_tiling_on_sc=False),
    )(_emb_body)
    return fn(idx, table)


def kernel(inputs, table):
    idx = inputs.reshape(_NW, _NCH, _CH).astype(jnp.int32)
    out = _embedding_lookup(idx, table)
    return out.reshape(_B, _L, _DIM)
